# baseline (speedup 1.0000x reference)
import functools

import jax
import jax.numpy as jnp
from jax import lax
from jax.experimental import pallas as pl
from jax.experimental.pallas import tpu as pltpu
from jax.experimental.pallas import tpu_sc as plsc

N = 10000
D = 128
H = 128
C = 64
E = 320000

NC = 2
NS = 16
K = 128
NCH = 160
EPAD = NS * NCH * K
NPAD = 10240
TRASH = NPAD - 1
RPT = NPAD // NS
NW = NC * NS
NCHW = EPAD // NW // K
RING = 5
WSC = 32
NSLOT = D // WSC

f32 = jnp.float32


def _sc_pass_body(ngroups, q_hbm, src_hbm, dst_hbm,
                  part_hbm, src_v, dst_v, srcg_v, rows_v, zb_v, acc_s,
                  *gsems):
    c = lax.axis_index("c")
    s = lax.axis_index("s")
    row0 = s * RPT

    pltpu.sync_copy(src_hbm.at[s], src_v)
    pltpu.sync_copy(dst_hbm.at[s], dst_v)

    zeros16 = jnp.zeros((16,), f32)

    def fill_zeros(r, carry):
        for j in range(WSC // 16):
            zb_v[r, pl.ds(j * 16, 16)] = zeros16
        return carry

    lax.fori_loop(0, RPT, fill_zeros, 0)

    for p in range(ngroups):
        grp = c * ngroups + p

        def scale_idx(r, carry):
            for j in range(K // 16):
                v = src_v[r, pl.ds(j * 16, 16)]
                srcg_v[r, pl.ds(j * 16, 16)] = v * NSLOT + grp
            return carry

        lax.fori_loop(0, NCH, scale_idx, 0)

        def gstart(b, ci):
            pltpu.async_copy(q_hbm.at[srcg_v.at[ci]], rows_v.at[b],
                             gsems[b])

        def gwait(b):
            pltpu.make_async_copy(q_hbm.at[pl.ds(0, K)], rows_v.at[b],
                                  gsems[b]).wait()

        pltpu.sync_copy(zb_v, acc_s.at[pl.ds(row0, RPT)])
        plsc.subcore_barrier()

        for b in range(RING):
            gstart(b, b)

        def outer(i, carry):
            for b in range(RING):
                ci = i * RING + b
                gwait(b)
                pltpu.sync_copy(rows_v.at[b], acc_s.at[dst_v.at[ci]],
                                add=True)

                @pl.when(ci + RING < NCH)
                def _():
                    gstart(b, ci + RING)
            return carry

        lax.fori_loop(0, NCH // RING, outer, 0)
        plsc.subcore_barrier()

        pltpu.sync_copy(acc_s.at[pl.ds(row0, RPT)],
                        part_hbm.at[pl.ds(row0, RPT),
                                    pl.ds(grp * WSC, WSC)])


def _sc_deg_body(dst_hbm, degpart_hbm, dst_v, dacc_t):
    c = lax.axis_index("c")
    s = lax.axis_index("s")
    wid = c * NS + s

    pltpu.sync_copy(dst_hbm.at[wid], dst_v)

    zeros16 = jnp.zeros((16,), f32)
    ones16 = jnp.ones((16,), f32)

    def zero_deg(r, carry):
        dacc_t[pl.ds(r * 16, 16)] = zeros16
        return carry

    lax.fori_loop(0, NPAD // 16, zero_deg, 0)

    def chunk(ci, carry):
        for j in range(K // 16):
            idx = dst_v[ci, pl.ds(j * 16, 16)]
            plsc.addupdate_scatter(dacc_t, [idx], ones16)
        return carry

    lax.fori_loop(0, NCHW, chunk, 0)
    pltpu.sync_copy(dacc_t, degpart_hbm.at[wid])


_SC_PARAMS = pltpu.CompilerParams(use_tc_tiling_on_sc=False,
                                  needs_layout_passes=False)


def _make_sc_pass(ngroups):
    mesh = plsc.VectorSubcoreMesh(core_axis_name="c", subcore_axis_name="s",
                                  num_cores=NC, num_subcores=NS)
    return pl.kernel(
        functools.partial(_sc_pass_body, ngroups),
        out_type=jax.ShapeDtypeStruct((NPAD, D), f32),
        mesh=mesh,
        scratch_types=(
            pltpu.VMEM((NCH, K), jnp.int32),
            pltpu.VMEM((NCH, K), jnp.int32),
            pltpu.VMEM((NCH, K), jnp.int32),
            pltpu.VMEM((RING, K, WSC), f32),
            pltpu.VMEM((RPT, WSC), f32),
            pltpu.VMEM_SHARED((NPAD, WSC), f32),
        ) + (pltpu.SemaphoreType.DMA,) * RING,
        compiler_params=_SC_PARAMS,
        name=f"sage_sc_agg_g{ngroups}",
    )


_sc_deg = pl.kernel(
    _sc_deg_body,
    out_type=jax.ShapeDtypeStruct((NW, NPAD), f32),
    mesh=plsc.VectorSubcoreMesh(core_axis_name="c", subcore_axis_name="s",
                                num_cores=NC, num_subcores=NS),
    scratch_types=(
        pltpu.VMEM((NCHW, K), jnp.int32),
        pltpu.VMEM((NPAD,), f32),
    ),
    compiler_params=_SC_PARAMS,
    name="sage_sc_deg",
)

_sc_agg1 = _make_sc_pass(2)
_sc_agg2 = _make_sc_pass(1)


RB = 1000
GRID = N // RB


def _a_body(x_ref, w_ref, b_ref, o_ref):
    o_ref[...] = (jnp.dot(x_ref[...], w_ref[...],
                          preferred_element_type=f32) + b_ref[...])


def _b_body(s1_ref, p_ref, dp_ref,
            wn1_ref, wn2_ref, ws2_ref, b2_ref, q2_ref, s2_ref):
    deg = jnp.maximum(jnp.sum(dp_ref[...], axis=1, keepdims=True), 1.0)
    neigh = p_ref[...] / deg
    h = s1_ref[...] + jnp.dot(neigh, wn1_ref[...], preferred_element_type=f32)
    h = jnp.maximum(h, 0.0)
    q2_ref[...] = jnp.dot(h, wn2_ref[...], preferred_element_type=f32)
    s2_ref[...] = (jnp.dot(h, ws2_ref[...], preferred_element_type=f32)
                   + b2_ref[...])


def _c_body(s2_ref, p_ref, dp_ref, o_ref):
    deg = jnp.maximum(jnp.sum(dp_ref[...], axis=1, keepdims=True), 1.0)
    o_ref[...] = s2_ref[...] + p_ref[:, :C] / deg


def _rows(shape):
    return pl.BlockSpec((RB,) + shape[1:], lambda i: (i,) + (0,) * (len(shape) - 1))


def _full(shape):
    return pl.BlockSpec(shape, lambda i: (0,) * len(shape))


_tc_a = pl.pallas_call(
    _a_body,
    grid=(GRID,),
    in_specs=[_rows((N, D)), _full((D, H)), _full((1, H))],
    out_specs=_rows((N, H)),
    out_shape=jax.ShapeDtypeStruct((N, H), f32),
)

_tc_b = pl.pallas_call(
    _b_body,
    grid=(GRID,),
    in_specs=[_rows((N, H)),
              _rows((NPAD, D)),
              _rows((NPAD, NW)),
              _full((D, H)),
              _full((H, D)), _full((H, C)), _full((1, C))],
    out_specs=(_rows((N, D)), _rows((N, C))),
    out_shape=(jax.ShapeDtypeStruct((N, D), f32),
               jax.ShapeDtypeStruct((N, C), f32)),
)

_tc_c = pl.pallas_call(
    _c_body,
    grid=(GRID,),
    in_specs=[_rows((N, C)),
              _rows((NPAD, D)),
              _rows((NPAD, NW))],
    out_specs=_rows((N, C)),
    out_shape=jax.ShapeDtypeStruct((N, C), f32),
)


def kernel(x, edge_index, W_self1, W_neigh1, b1, W_self2, W_neigh2, b2):
    srcf = jnp.concatenate(
        [edge_index[0].astype(jnp.int32),
         jnp.zeros((EPAD - E,), jnp.int32)])
    dstf = jnp.concatenate(
        [edge_index[1].astype(jnp.int32),
         jnp.full((EPAD - E,), TRASH, jnp.int32)])
    src = srcf.reshape(NS, NCH, K)
    dst = dstf.reshape(NS, NCH, K)
    dstw = dstf.reshape(NW, NCHW, K)

    wn2p = jnp.pad(W_neigh2, ((0, 0), (0, D - C)))

    s1 = _tc_a(x, W_self1, b1.reshape(1, H))
    degparts = _sc_deg(dstw)
    parts1 = _sc_agg1(x.reshape(NSLOT * N, WSC), src, dst)
    degT = degparts.T
    q2, s2 = _tc_b(s1, parts1, degT,
                   W_neigh1, wn2p, W_self2, b2.reshape(1, C))
    parts2 = _sc_agg2(q2.reshape(NSLOT * N, WSC), src, dst)
    return _tc_c(s2, parts2, degT)


# speedup vs baseline: 2.8428x; 2.8428x over previous
import functools

import jax
import jax.numpy as jnp
from jax import lax
from jax.experimental import pallas as pl
from jax.experimental.pallas import tpu as pltpu
from jax.experimental.pallas import tpu_sc as plsc

N = 10000
D = 128
H = 128
C = 64
E = 320000

NC = 2
NS = 16
K = 80
NCH = E // NS // K
NPAD = 10240
RPT = NPAD // NS
NW = NC * NS
NCHW = E // NW // K
RING = 10
WSC = 32
NSLOT = D // WSC

f32 = jnp.float32


def _sc_pass_body(ngroups, q_hbm, src_hbm, dst_hbm,
                  part_hbm, src_v, dst_v, srcg_v, rows_v, zb_v, acc_s,
                  *gsems):
    c = lax.axis_index("c")
    s = lax.axis_index("s")
    row0 = s * RPT

    pltpu.sync_copy(src_hbm.at[s], src_v)
    pltpu.sync_copy(dst_hbm.at[s], dst_v)

    zeros16 = jnp.zeros((16,), f32)

    def fill_zeros(r, carry):
        for j in range(WSC // 16):
            zb_v[r, pl.ds(j * 16, 16)] = zeros16
        return carry

    lax.fori_loop(0, RPT, fill_zeros, 0)

    for p in range(ngroups):
        grp = c * ngroups + p

        def scale_idx(r, carry):
            for j in range(K // 16):
                v = src_v[r, pl.ds(j * 16, 16)]
                srcg_v[r, pl.ds(j * 16, 16)] = v * NSLOT + grp
            return carry

        lax.fori_loop(0, NCH, scale_idx, 0)

        def gstart(b, ci):
            pltpu.async_copy(q_hbm.at[srcg_v.at[ci]], rows_v.at[b],
                             gsems[b])

        def gwait(b):
            pltpu.make_async_copy(q_hbm.at[pl.ds(0, K)], rows_v.at[b],
                                  gsems[b]).wait()

        pltpu.sync_copy(zb_v, acc_s.at[pl.ds(row0, RPT)])
        plsc.subcore_barrier()

        for b in range(RING):
            gstart(b, b)

        def outer(i, carry):
            for b in range(RING):
                ci = i * RING + b
                gwait(b)
                pltpu.sync_copy(rows_v.at[b], acc_s.at[dst_v.at[ci]],
                                add=True)

                @pl.when(ci + RING < NCH)
                def _():
                    gstart(b, ci + RING)
            return carry

        lax.fori_loop(0, NCH // RING, outer, 0)
        plsc.subcore_barrier()

        pltpu.sync_copy(acc_s.at[pl.ds(row0, RPT)],
                        part_hbm.at[pl.ds(row0, RPT),
                                    pl.ds(grp * WSC, WSC)])


def _sc_deg_body(dst_hbm, degpart_hbm, dst_v, dacc_t):
    c = lax.axis_index("c")
    s = lax.axis_index("s")
    wid = c * NS + s

    pltpu.sync_copy(dst_hbm.at[wid], dst_v)

    zeros16 = jnp.zeros((16,), f32)
    ones16 = jnp.ones((16,), f32)

    def zero_deg(r, carry):
        dacc_t[pl.ds(r * 16, 16)] = zeros16
        return carry

    lax.fori_loop(0, NPAD // 16, zero_deg, 0)

    def chunk(ci, carry):
        for j in range(K // 16):
            idx = dst_v[ci, pl.ds(j * 16, 16)]
            plsc.addupdate_scatter(dacc_t, [idx], ones16)
        return carry

    lax.fori_loop(0, NCHW, chunk, 0)
    pltpu.sync_copy(dacc_t, degpart_hbm.at[wid])


_SC_PARAMS = pltpu.CompilerParams(use_tc_tiling_on_sc=False,
                                  needs_layout_passes=False)


def _make_sc_pass(ngroups):
    mesh = plsc.VectorSubcoreMesh(core_axis_name="c", subcore_axis_name="s",
                                  num_cores=NC, num_subcores=NS)
    return pl.kernel(
        functools.partial(_sc_pass_body, ngroups),
        out_type=jax.ShapeDtypeStruct((NPAD, D), f32),
        mesh=mesh,
        scratch_types=(
            pltpu.VMEM((NCH, K), jnp.int32),
            pltpu.VMEM((NCH, K), jnp.int32),
            pltpu.VMEM((NCH, K), jnp.int32),
            pltpu.VMEM((RING, K, WSC), f32),
            pltpu.VMEM((RPT, WSC), f32),
            pltpu.VMEM_SHARED((NPAD, WSC), f32),
        ) + (pltpu.SemaphoreType.DMA,) * RING,
        compiler_params=_SC_PARAMS,
        name=f"sage_sc_agg_g{ngroups}",
    )


_sc_deg = pl.kernel(
    _sc_deg_body,
    out_type=jax.ShapeDtypeStruct((NW, NPAD), f32),
    mesh=plsc.VectorSubcoreMesh(core_axis_name="c", subcore_axis_name="s",
                                num_cores=NC, num_subcores=NS),
    scratch_types=(
        pltpu.VMEM((NCHW, K), jnp.int32),
        pltpu.VMEM((NPAD,), f32),
    ),
    compiler_params=_SC_PARAMS,
    name="sage_sc_deg",
)

_sc_agg1 = _make_sc_pass(2)
_sc_agg2 = _make_sc_pass(1)


RB = 1000
GRID = N // RB


def _a_body(x_ref, w_ref, b_ref, o_ref):
    o_ref[...] = (jnp.dot(x_ref[...], w_ref[...],
                          preferred_element_type=f32) + b_ref[...])


def _b_body(s1_ref, p_ref, dp_ref,
            wn1_ref, wn2_ref, ws2_ref, b2_ref, q2_ref, s2_ref):
    deg = jnp.maximum(jnp.sum(dp_ref[...], axis=1, keepdims=True), 1.0)
    neigh = p_ref[...] / deg
    h = s1_ref[...] + jnp.dot(neigh, wn1_ref[...], preferred_element_type=f32)
    h = jnp.maximum(h, 0.0)
    q2_ref[...] = jnp.dot(h, wn2_ref[...], preferred_element_type=f32)
    s2_ref[...] = (jnp.dot(h, ws2_ref[...], preferred_element_type=f32)
                   + b2_ref[...])


def _c_body(s2_ref, p_ref, dp_ref, o_ref):
    deg = jnp.maximum(jnp.sum(dp_ref[...], axis=1, keepdims=True), 1.0)
    o_ref[...] = s2_ref[...] + p_ref[:, :C] / deg


def _rows(shape):
    return pl.BlockSpec((RB,) + shape[1:], lambda i: (i,) + (0,) * (len(shape) - 1))


def _full(shape):
    return pl.BlockSpec(shape, lambda i: (0,) * len(shape))


_tc_a = pl.pallas_call(
    _a_body,
    grid=(GRID,),
    in_specs=[_rows((N, D)), _full((D, H)), _full((1, H))],
    out_specs=_rows((N, H)),
    out_shape=jax.ShapeDtypeStruct((N, H), f32),
)

_tc_b = pl.pallas_call(
    _b_body,
    grid=(GRID,),
    in_specs=[_rows((N, H)),
              _rows((NPAD, D)),
              _rows((NPAD, NW)),
              _full((D, H)),
              _full((H, D)), _full((H, C)), _full((1, C))],
    out_specs=(_rows((N, D)), _rows((N, C))),
    out_shape=(jax.ShapeDtypeStruct((N, D), f32),
               jax.ShapeDtypeStruct((N, C), f32)),
)

_tc_c = pl.pallas_call(
    _c_body,
    grid=(GRID,),
    in_specs=[_rows((N, C)),
              _rows((NPAD, D)),
              _rows((NPAD, NW))],
    out_specs=_rows((N, C)),
    out_shape=jax.ShapeDtypeStruct((N, C), f32),
)


def kernel(x, edge_index, W_self1, W_neigh1, b1, W_self2, W_neigh2, b2):
    src = edge_index[0].astype(jnp.int32).reshape(NS, NCH, K)
    dst = edge_index[1].astype(jnp.int32).reshape(NS, NCH, K)
    dstw = edge_index[1].astype(jnp.int32).reshape(NW, NCHW, K)

    wn2p = jnp.pad(W_neigh2, ((0, 0), (0, D - C)))

    s1 = _tc_a(x, W_self1, b1.reshape(1, H))
    degparts = _sc_deg(dstw)
    parts1 = _sc_agg1(x.reshape(NSLOT * N, WSC), src, dst)
    degT = degparts.T
    q2, s2 = _tc_b(s1, parts1, degT,
                   W_neigh1, wn2p, W_self2, b2.reshape(1, C))
    parts2 = _sc_agg2(q2.reshape(NSLOT * N, WSC), src, dst)
    return _tc_c(s2, parts2, degT)
